# Initial kernel scaffold; baseline (speedup 1.0000x reference)
import jax
import jax.numpy as jnp
from jax.experimental import pallas as pl


def kernel(x_dict, edge_index, edge_attr, params):
    raise NotImplementedError("write your pallas kernel here")


# speedup vs baseline: 2.1215x; 2.1215x over previous
import functools

import jax
import jax.numpy as jnp
from jax.experimental import pallas as pl

_NODE_TYPES = ('pfas_sites', 'sw_stations', 'gw_wells')
_EDGE_TYPES = (
    ('pfas_sites', 'gw_wells'),
    ('pfas_sites', 'sw_stations'),
    ('sw_stations', 'pfas_sites'),
    ('sw_stations', 'gw_wells'),
    ('gw_wells', 'sw_stations'),
    ('gw_wells', 'gw_wells'),
    ('gw_wells', 'pfas_sites'),
)


def _ek(e):
    return e[0] + '->' + e[1]


def _bn(x, g, b, eps=1e-5):
    mu = jnp.mean(x, 0)
    var = jnp.var(x, 0)
    return g * (x - mu) / jnp.sqrt(var + eps) + b


def _seg_sum(vals, col, n):
    return jax.ops.segment_sum(vals, col, num_segments=n)


def _conv_edge(xd_src, xd_dst, row, col, cnt, has_w, eat_w, p):
    s = _seg_sum(xd_src[row], col, xd_dst.shape[0])
    mean = s / jnp.maximum(cnt, 1.0)[:, None]
    out = mean @ p['lin_l_w'] + p['lin_l_b'] + xd_dst @ p['lin_r_w']
    att_w = p['att_w']
    w_top, w_bot = att_w[:out.shape[1]], att_w[out.shape[1]:]
    score = out @ w_top + eat_w @ w_bot + p['att_b']
    attn = jax.nn.sigmoid(score)
    out = out + jnp.where(has_w[:, None], attn * eat_w, 0.0)
    out = _bn(out, p['bn_g'], p['bn_b'])
    return jax.nn.relu(out + out)


def kernel(x_dict, edge_index, edge_attr, params):
    n_nodes = {t: x_dict[t].shape[0] for t in _NODE_TYPES}

    xd = {}
    for t in _NODE_TYPES:
        q = params['node_red'][t]
        h = x_dict[t] @ q['w'] + q['b']
        xd[t] = jax.nn.relu(_bn(h, q['g'], q['be']))

    cnt = {}
    has_w = {}
    eat_win = {}
    ead_win = {}
    for e in _EDGE_TYPES:
        k = _ek(e)
        ei = edge_index[k]
        row, col = ei[0], ei[1]
        nd = n_nodes[e[1]]
        E = row.shape[0]
        cnt[k] = _seg_sum(jnp.ones((E,), jnp.float32), col, nd)
        winner = jnp.full((nd,), -1, jnp.int32).at[col].max(
            jnp.arange(E, dtype=jnp.int32))
        has_w[k] = winner >= 0
        wsafe = jnp.maximum(winner, 0)
        ea = edge_attr[k]
        ea_w = ea[wsafe]
        q = params['edge_red'][k]
        mu_ea = jnp.mean(ea, 0)
        G = ea.T @ ea / E
        cov = G - mu_ea[:, None] * mu_ea[None, :]
        mean_h = mu_ea @ q['w'] + q['b']
        var_h = jnp.sum(q['w'] * (cov @ q['w']), 0)
        h_w = ea_w @ q['w'] + q['b']
        ead_win[k] = jax.nn.relu(
            q['g'] * (h_w - mean_h) / jnp.sqrt(var_h + 1e-5) + q['be'])

    x = xd
    for layer in ('conv1', 'conv2'):
        pl_ = params[layer]
        acc = {t: None for t in _NODE_TYPES}
        for e in _EDGE_TYPES:
            k = _ek(e)
            p = pl_[k]
            ei = edge_index[k]
            eat_w = ead_win[k] @ p['et_w'] + p['et_b']
            o = _conv_edge(x[e[0]], x[e[1]], ei[0], ei[1],
                           cnt[k], has_w[k], eat_w, p)
            acc[e[1]] = o if acc[e[1]] is None else acc[e[1]] + o
        for t in _NODE_TYPES:
            p = pl_['self:' + t]
            o = x[t] @ (p['lin_l_w'] + p['lin_r_w']) + p['lin_l_b']
            acc[t] = o if acc[t] is None else acc[t] + o
        x = {t: jax.nn.relu(acc[t]) for t in _NODE_TYPES}

    w, b = params['linear']['w'], params['linear']['b']
    a = params['prelu']

    def prelu(v):
        return jnp.where(v >= 0, v, a * v)

    gw = prelu(x['gw_wells'] @ w + b)
    sw = prelu(x['sw_stations'] @ w + b)
    return x['pfas_sites'], sw, gw


# speedup vs baseline: 2.5811x; 1.2167x over previous
import functools

import jax
import jax.numpy as jnp
from jax import lax
from jax.experimental import pallas as pl
from jax.experimental.pallas import tpu as pltpu
from jax.experimental.pallas import tpu_sc as plsc

_NC, _NS, _L = 2, 16, 16
_NW = _NC * _NS
_K = 128
_ZR = 64
_SPMEM_BUDGET = 7 * 1024 * 1024


def _sc_mesh():
    return plsc.VectorSubcoreMesh(core_axis_name="c", subcore_axis_name="s")


@functools.cache
def _segsum_kernel(c, nd_pad, e_pad):
    epw = e_pad // _NW
    nchunks = epw // _K
    rps = nd_pad // _NS

    @functools.partial(
        pl.kernel,
        out_type=jax.ShapeDtypeStruct((_NC, nd_pad, c), jnp.float32),
        mesh=_sc_mesh(),
        scratch_types=[
            pltpu.VMEM((_K,), jnp.int32),
            pltpu.VMEM((_K,), jnp.int32),
            pltpu.VMEM((_K, c), jnp.float32),
            pltpu.VMEM((_ZR, c), jnp.float32),
            pltpu.VMEM_SHARED((nd_pad, c), jnp.float32),
            pltpu.SemaphoreType.DMA,
        ],
        compiler_params=pltpu.CompilerParams(use_tc_tiling_on_sc=False),
    )
    def k(x_hbm, row_hbm, col_hbm, out_hbm, row_v, col_v, gbuf, ztile, acc, sem):
        cid = lax.axis_index("c")
        sid = lax.axis_index("s")
        zv = jnp.zeros((_L,), jnp.float32)

        @pl.loop(0, _ZR)
        def _(i):
            @pl.loop(0, c, step=_L)
            def _(j):
                ztile[i, pl.ds(j, _L)] = zv

        rbase = sid * rps

        @pl.loop(0, rps, step=_ZR)
        def _(r):
            pltpu.sync_copy(ztile, acc.at[pl.ds(rbase + r, _ZR)])

        plsc.subcore_barrier()

        wid = sid * _NC + cid
        base = wid * epw

        @pl.loop(0, nchunks)
        def _(i):
            off = base + i * _K
            pltpu.sync_copy(row_hbm.at[pl.ds(off, _K)], row_v)
            pltpu.async_copy(x_hbm.at[row_v], gbuf, sem).wait()
            pltpu.sync_copy(col_hbm.at[pl.ds(off, _K)], col_v)
            pltpu.sync_copy(gbuf, acc.at[col_v], add=True)

        plsc.subcore_barrier()
        pltpu.sync_copy(acc.at[pl.ds(rbase, rps)],
                        out_hbm.at[cid].at[pl.ds(rbase, rps)])

    return k


def _pad1(a, n, fill):
    if n == a.shape[0]:
        return a
    return jnp.concatenate(
        [a, jnp.full((n - a.shape[0],), fill, a.dtype)])


def _ceil_to(x, m):
    return -(-x // m) * m


def _sc_segsum(x, row, col, nd):
    ns, c = x.shape
    e_pad = _ceil_to(row.shape[0], _NW * _K)
    nd_pad = _ceil_to(nd + 1, _ZR * _NS)
    rowp = _pad1(row, e_pad, 0)
    colp = _pad1(col, e_pad, nd)
    cw = c
    while nd_pad * cw * 4 > _SPMEM_BUDGET:
        cw //= 2
    parts = []
    for i in range(0, c, cw):
        out = _segsum_kernel(cw, nd_pad, e_pad)(x[:, i:i + cw], rowp, colp)
        parts.append(out[0, :nd] + out[1, :nd])
    return parts[0] if len(parts) == 1 else jnp.concatenate(parts, -1)

_NODE_TYPES = ('pfas_sites', 'sw_stations', 'gw_wells')
_EDGE_TYPES = (
    ('pfas_sites', 'gw_wells'),
    ('pfas_sites', 'sw_stations'),
    ('sw_stations', 'pfas_sites'),
    ('sw_stations', 'gw_wells'),
    ('gw_wells', 'sw_stations'),
    ('gw_wells', 'gw_wells'),
    ('gw_wells', 'pfas_sites'),
)


def _ek(e):
    return e[0] + '->' + e[1]


def _bn(x, g, b, eps=1e-5):
    mu = jnp.mean(x, 0)
    var = jnp.var(x, 0)
    return g * (x - mu) / jnp.sqrt(var + eps) + b


def _seg_sum(vals, col, n):
    return jax.ops.segment_sum(vals, col, num_segments=n)


def _conv_edge(xd_src, xd_dst, row, col, cnt, has_w, eat_w, p):
    s = _sc_segsum(xd_src, row, col, xd_dst.shape[0])
    mean = s / jnp.maximum(cnt, 1.0)[:, None]
    out = mean @ p['lin_l_w'] + p['lin_l_b'] + xd_dst @ p['lin_r_w']
    att_w = p['att_w']
    w_top, w_bot = att_w[:out.shape[1]], att_w[out.shape[1]:]
    score = out @ w_top + eat_w @ w_bot + p['att_b']
    attn = jax.nn.sigmoid(score)
    out = out + jnp.where(has_w[:, None], attn * eat_w, 0.0)
    out = _bn(out, p['bn_g'], p['bn_b'])
    return jax.nn.relu(out + out)


def kernel(x_dict, edge_index, edge_attr, params):
    n_nodes = {t: x_dict[t].shape[0] for t in _NODE_TYPES}

    xd = {}
    for t in _NODE_TYPES:
        q = params['node_red'][t]
        h = x_dict[t] @ q['w'] + q['b']
        xd[t] = jax.nn.relu(_bn(h, q['g'], q['be']))

    cnt = {}
    has_w = {}
    eat_win = {}
    ead_win = {}
    for e in _EDGE_TYPES:
        k = _ek(e)
        ei = edge_index[k]
        row, col = ei[0], ei[1]
        nd = n_nodes[e[1]]
        E = row.shape[0]
        cnt[k] = _seg_sum(jnp.ones((E,), jnp.float32), col, nd)
        winner = jnp.full((nd,), -1, jnp.int32).at[col].max(
            jnp.arange(E, dtype=jnp.int32))
        has_w[k] = winner >= 0
        wsafe = jnp.maximum(winner, 0)
        ea = edge_attr[k]
        ea_w = ea[wsafe]
        q = params['edge_red'][k]
        mu_ea = jnp.mean(ea, 0)
        G = ea.T @ ea / E
        cov = G - mu_ea[:, None] * mu_ea[None, :]
        mean_h = mu_ea @ q['w'] + q['b']
        var_h = jnp.sum(q['w'] * (cov @ q['w']), 0)
        h_w = ea_w @ q['w'] + q['b']
        ead_win[k] = jax.nn.relu(
            q['g'] * (h_w - mean_h) / jnp.sqrt(var_h + 1e-5) + q['be'])

    x = xd
    for layer in ('conv1', 'conv2'):
        pl_ = params[layer]
        acc = {t: None for t in _NODE_TYPES}
        for e in _EDGE_TYPES:
            k = _ek(e)
            p = pl_[k]
            ei = edge_index[k]
            eat_w = ead_win[k] @ p['et_w'] + p['et_b']
            o = _conv_edge(x[e[0]], x[e[1]], ei[0], ei[1],
                           cnt[k], has_w[k], eat_w, p)
            acc[e[1]] = o if acc[e[1]] is None else acc[e[1]] + o
        for t in _NODE_TYPES:
            p = pl_['self:' + t]
            o = x[t] @ (p['lin_l_w'] + p['lin_r_w']) + p['lin_l_b']
            acc[t] = o if acc[t] is None else acc[t] + o
        x = {t: jax.nn.relu(acc[t]) for t in _NODE_TYPES}

    w, b = params['linear']['w'], params['linear']['b']
    a = params['prelu']

    def prelu(v):
        return jnp.where(v >= 0, v, a * v)

    gw = prelu(x['gw_wells'] @ w + b)
    sw = prelu(x['sw_stations'] @ w + b)
    return x['pfas_sites'], sw, gw
